# baseline (speedup 1.0000x reference)
import jax
import jax.numpy as jnp
from jax import lax
from jax.experimental import pallas as pl
from jax.experimental.pallas import tpu as pltpu
from jax.experimental.pallas import tpu_sc as plsc

_BS = 512
_K = 3072
_NC = 2
_NS = 16
_NW = _NC * _NS
_L = 16
_C = 16
_ROW = 1024


def _add_kernel(x_ref, t_ref, o_ref):
    o_ref[...] = x_ref[...] + t_ref[...][None, :, :]


def _tc_call(x, pe):
    B, S, D = x.shape
    k0 = _K // _BS
    return pl.pallas_call(
        _add_kernel,
        grid=((S - _K) // _BS,),
        in_specs=[
            pl.BlockSpec((B, _BS, D), lambda i: (0, i + k0, 0)),
            pl.BlockSpec((_BS, D), lambda i: (i + k0, 0)),
        ],
        out_specs=pl.BlockSpec((B, _BS, D), lambda i: (0, i + k0, 0)),
        out_shape=jax.ShapeDtypeStruct((B, S, D), x.dtype),
        compiler_params=pltpu.CompilerParams(
            dimension_semantics=("parallel",),
        ),
    )(x, pe)


def _sc_body(x_hbm, t_hbm, o_hbm,
             xb0, xb1, tb0, tb1, ob0, ob1,
             sx0, sx1, st0, st1, so0, so1):
    B = x_hbm.shape[0]
    rows_per_w = (B * _K) // _NW
    n_chunks = rows_per_w // _C
    wpb = _NW // B
    w = lax.axis_index("c") * _NS + lax.axis_index("s")
    bi = w // wpb
    seq0 = (w % wpb) * rows_per_w

    xb = (xb0, xb1)
    tb = (tb0, tb1)
    ob = (ob0, ob1)
    sx = (sx0, sx1)
    st = (st0, st1)
    so = (so0, so1)

    def fire_in(g, b):
        r = seq0 + g * _C
        pltpu.make_async_copy(
            x_hbm.at[bi, pl.ds(r, _C), :], xb[b], sx[b]).start()
        pltpu.make_async_copy(
            t_hbm.at[pl.ds(r, _C), :], tb[b], st[b]).start()

    def wait_in(b):
        pltpu.make_async_copy(
            x_hbm.at[bi, pl.ds(seq0, _C), :], xb[b], sx[b]).wait()
        pltpu.make_async_copy(
            t_hbm.at[pl.ds(seq0, _C), :], tb[b], st[b]).wait()

    def fire_out(g, b):
        r = seq0 + g * _C
        pltpu.make_async_copy(
            ob[b], o_hbm.at[bi, pl.ds(r, _C), :], so[b]).start()

    def wait_out(b):
        pltpu.make_async_copy(
            x_hbm.at[bi, pl.ds(seq0, _C), :], ob[b], so[b]).wait()

    def compute(b):
        xr, tr, orr = xb[b], tb[b], ob[b]

        def cbody(k, carry):
            base = pl.multiple_of(k * (_L * 8), _L * 8)
            for u in range(8):
                s = base + u * _L
                for r in range(_C):
                    orr[r, pl.ds(s, _L)] = (
                        xr[r, pl.ds(s, _L)] + tr[r, pl.ds(s, _L)])
            return carry

        lax.fori_loop(0, _ROW // (_L * 8), cbody, 0)

    fire_in(0, 0)
    fire_in(1, 1)

    def body(j, carry):
        for b in range(2):
            g = 2 * j + b
            wait_in(b)

            @pl.when(j > 0)
            def _():
                wait_out(b)

            compute(b)
            fire_out(g, b)
            fire_in(g + 2, b)
        return carry

    lax.fori_loop(0, n_chunks // 2 - 1, body, 0)

    for b in range(2):
        wait_in(b)
        wait_out(b)
        compute(b)
        fire_out(n_chunks - 2 + b, b)
    for b in range(2):
        wait_out(b)


def _sc_call(x, table):
    B, S, D = x.shape
    mesh = plsc.VectorSubcoreMesh(core_axis_name="c", subcore_axis_name="s")
    k = pl.kernel(
        _sc_body,
        out_type=jax.ShapeDtypeStruct((B, _K, D), x.dtype),
        mesh=mesh,
        scratch_types=(
            [pltpu.VMEM((_C, _ROW), jnp.float32)] * 6
            + [pltpu.SemaphoreType.DMA] * 6
        ),
        compiler_params=pltpu.CompilerParams(use_tc_tiling_on_sc=True),
    )
    return k(x, table)


def kernel(x, table, fea_ind):
    B, S, D = x.shape
    pe = jax.lax.slice(table, (0, 0), (S, D))
    sc_out = _sc_call(x, pe)
    tc_out = _tc_call(x, pe)
    return lax.dynamic_update_slice(tc_out, sc_out, (0, 0, 0))


# speedup vs baseline: 1.7454x; 1.7454x over previous
import jax
import jax.numpy as jnp
from jax.experimental import pallas as pl
from jax.experimental.pallas import tpu as pltpu

_BS = 512


def _add_kernel(x_ref, t_ref, o_ref):
    o_ref[...] = x_ref[...] + t_ref[...][None, :, :]


def kernel(x, table, fea_ind):
    B, S, D = x.shape
    pe = jax.lax.slice(table, (0, 0), (S, D))
    grid = (S // _BS,)
    return pl.pallas_call(
        _add_kernel,
        grid=grid,
        in_specs=[
            pl.BlockSpec((B, _BS, D), lambda i: (0, i, 0)),
            pl.BlockSpec((_BS, D), lambda i: (i, 0)),
        ],
        out_specs=pl.BlockSpec((B, _BS, D), lambda i: (0, i, 0)),
        out_shape=jax.ShapeDtypeStruct((B, S, D), x.dtype),
        compiler_params=pltpu.CompilerParams(
            dimension_semantics=("parallel",),
        ),
    )(x, pe)


# speedup vs baseline: 1.7514x; 1.0034x over previous
import jax
import jax.numpy as jnp
from jax import lax
from jax.experimental import pallas as pl
from jax.experimental.pallas import tpu as pltpu

_BS = 256
_NB = 4


def _body(x_hbm, t_hbm, o_hbm, xb, tb, ob, six, sit, so):
    B, S, D = x_hbm.shape
    nch = S // _BS

    def fire_in(g):
        slot = lax.rem(g, _NB)
        r = g * _BS
        pltpu.make_async_copy(
            x_hbm.at[:, pl.ds(r, _BS), :], xb.at[slot], six.at[slot]).start()
        pltpu.make_async_copy(
            t_hbm.at[pl.ds(r, _BS), :], tb.at[slot], sit.at[slot]).start()

    def wait_in(slot):
        pltpu.make_async_copy(
            x_hbm.at[:, pl.ds(0, _BS), :], xb.at[slot], six.at[slot]).wait()
        pltpu.make_async_copy(
            t_hbm.at[pl.ds(0, _BS), :], tb.at[slot], sit.at[slot]).wait()

    def fire_out(g):
        slot = lax.rem(g, _NB)
        r = g * _BS
        pltpu.make_async_copy(
            ob.at[slot], o_hbm.at[:, pl.ds(r, _BS), :], so.at[slot]).start()

    def wait_out(slot):
        pltpu.make_async_copy(
            x_hbm.at[:, pl.ds(0, _BS), :], ob.at[slot], so.at[slot]).wait()

    for g in range(_NB):
        fire_in(g)

    def loop(g, carry):
        slot = lax.rem(g, _NB)
        wait_in(slot)

        @pl.when(g >= _NB)
        def _():
            wait_out(slot)

        ob[slot] = xb[slot] + tb[slot][None]
        fire_out(g)

        @pl.when(g + _NB < nch)
        def _():
            fire_in(g + _NB)

        return carry

    lax.fori_loop(0, nch, loop, 0)

    for s in range(_NB):
        wait_out(s)


def kernel(x, table, fea_ind):
    B, S, D = x.shape
    pe = jax.lax.slice(table, (0, 0), (S, D))
    return pl.pallas_call(
        _body,
        in_specs=[
            pl.BlockSpec(memory_space=pl.ANY),
            pl.BlockSpec(memory_space=pl.ANY),
        ],
        out_specs=pl.BlockSpec(memory_space=pl.ANY),
        out_shape=jax.ShapeDtypeStruct((B, S, D), x.dtype),
        scratch_shapes=[
            pltpu.VMEM((_NB, B, _BS, D), jnp.float32),
            pltpu.VMEM((_NB, _BS, D), jnp.float32),
            pltpu.VMEM((_NB, B, _BS, D), jnp.float32),
            pltpu.SemaphoreType.DMA((_NB,)),
            pltpu.SemaphoreType.DMA((_NB,)),
            pltpu.SemaphoreType.DMA((_NB,)),
        ],
    )(x, pe)


# speedup vs baseline: 1.7533x; 1.0011x over previous
import jax
import jax.numpy as jnp
from jax import lax
from jax.experimental import pallas as pl
from jax.experimental.pallas import tpu as pltpu

_BS = 128
_NB = 8


def _body(x_hbm, t_hbm, o_hbm, xb, tb, ob, six, sit, so):
    B, S, D = x_hbm.shape
    nch = S // _BS

    def fire_in(g):
        slot = lax.rem(g, _NB)
        r = g * _BS
        pltpu.make_async_copy(
            x_hbm.at[:, pl.ds(r, _BS), :], xb.at[slot], six.at[slot]).start()
        pltpu.make_async_copy(
            t_hbm.at[pl.ds(r, _BS), :], tb.at[slot], sit.at[slot]).start()

    def wait_in(slot):
        pltpu.make_async_copy(
            x_hbm.at[:, pl.ds(0, _BS), :], xb.at[slot], six.at[slot]).wait()
        pltpu.make_async_copy(
            t_hbm.at[pl.ds(0, _BS), :], tb.at[slot], sit.at[slot]).wait()

    def fire_out(g):
        slot = lax.rem(g, _NB)
        r = g * _BS
        pltpu.make_async_copy(
            ob.at[slot], o_hbm.at[:, pl.ds(r, _BS), :], so.at[slot]).start()

    def wait_out(slot):
        pltpu.make_async_copy(
            x_hbm.at[:, pl.ds(0, _BS), :], ob.at[slot], so.at[slot]).wait()

    for g in range(_NB):
        fire_in(g)

    def loop(g, carry):
        slot = lax.rem(g, _NB)
        wait_in(slot)

        @pl.when(g >= _NB)
        def _():
            wait_out(slot)

        ob[slot] = xb[slot] + tb[slot][None]
        fire_out(g)

        @pl.when(g + _NB < nch)
        def _():
            fire_in(g + _NB)

        return carry

    lax.fori_loop(0, nch, loop, 0)

    for s in range(_NB):
        wait_out(s)


def kernel(x, table, fea_ind):
    B, S, D = x.shape
    pe = jax.lax.slice(table, (0, 0), (S, D))
    return pl.pallas_call(
        _body,
        in_specs=[
            pl.BlockSpec(memory_space=pl.ANY),
            pl.BlockSpec(memory_space=pl.ANY),
        ],
        out_specs=pl.BlockSpec(memory_space=pl.ANY),
        out_shape=jax.ShapeDtypeStruct((B, S, D), x.dtype),
        scratch_shapes=[
            pltpu.VMEM((_NB, B, _BS, D), jnp.float32),
            pltpu.VMEM((_NB, _BS, D), jnp.float32),
            pltpu.VMEM((_NB, B, _BS, D), jnp.float32),
            pltpu.SemaphoreType.DMA((_NB,)),
            pltpu.SemaphoreType.DMA((_NB,)),
            pltpu.SemaphoreType.DMA((_NB,)),
        ],
    )(x, pe)


# speedup vs baseline: 1.7576x; 1.0025x over previous
import jax
import jax.numpy as jnp
from jax import lax
from jax.experimental import pallas as pl
from jax.experimental.pallas import tpu as pltpu

_BS = 256
_NB = 4


def _body(x_hbm, t_hbm, o_hbm, xb, tb, ob, six, sit, so):
    B, S, D = x_hbm.shape
    nch = S // _BS

    def fire_in(g):
        slot = lax.rem(g, _NB)
        r = g * _BS
        pltpu.make_async_copy(
            x_hbm.at[:, pl.ds(r, _BS), :], xb.at[slot], six.at[slot]).start()
        pltpu.make_async_copy(
            t_hbm.at[pl.ds(r, _BS), :], tb.at[slot], sit.at[slot]).start()

    def wait_in(slot):
        pltpu.make_async_copy(
            x_hbm.at[:, pl.ds(0, _BS), :], xb.at[slot], six.at[slot]).wait()
        pltpu.make_async_copy(
            t_hbm.at[pl.ds(0, _BS), :], tb.at[slot], sit.at[slot]).wait()

    def fire_out(g):
        slot = lax.rem(g, _NB)
        r = g * _BS
        pltpu.make_async_copy(
            ob.at[slot], o_hbm.at[:, pl.ds(r, _BS), :], so.at[slot]).start()

    def wait_out(slot):
        pltpu.make_async_copy(
            x_hbm.at[:, pl.ds(0, _BS), :], ob.at[slot], so.at[slot]).wait()

    for g in range(_NB):
        fire_in(g)

    def loop(g, carry):
        slot = lax.rem(g, _NB)
        wait_in(slot)

        @pl.when(g >= _NB)
        def _():
            wait_out(slot)

        ob[slot] = xb[slot] + tb[slot][None]
        fire_out(g)

        @pl.when(g + _NB < nch)
        def _():
            fire_in(g + _NB)

        return carry

    lax.fori_loop(0, nch, loop, 0)

    for s in range(_NB):
        wait_out(s)


def kernel(x, table, fea_ind):
    B, S, D = x.shape
    pe = jax.lax.slice(table, (0, 0), (S, D))
    return pl.pallas_call(
        _body,
        in_specs=[
            pl.BlockSpec(memory_space=pl.ANY),
            pl.BlockSpec(memory_space=pl.ANY),
        ],
        out_specs=pl.BlockSpec(memory_space=pl.ANY),
        out_shape=jax.ShapeDtypeStruct((B, S, D), x.dtype),
        scratch_shapes=[
            pltpu.VMEM((_NB, B, _BS, D), jnp.float32),
            pltpu.VMEM((_NB, _BS, D), jnp.float32),
            pltpu.VMEM((_NB, B, _BS, D), jnp.float32),
            pltpu.SemaphoreType.DMA((_NB,)),
            pltpu.SemaphoreType.DMA((_NB,)),
            pltpu.SemaphoreType.DMA((_NB,)),
        ],
    )(x, pe)
